# baseline (speedup 1.0000x reference)
import functools

import jax
import jax.numpy as jnp
from jax import lax
from jax.experimental import pallas as pl
from jax.experimental.pallas import tpu as pltpu
from jax.experimental.pallas import tpu_sc as plsc

D_MODEL = 64
D_PAD = 128
BATCH = 16384
EPS = 1e-5

NUM_CORES = 2
NUM_SUBCORES = 16
NUM_WORKERS = NUM_CORES * NUM_SUBCORES
ROWS_PER_WORKER = BATCH // NUM_WORKERS
CHUNK = 128
NUM_CHUNKS = ROWS_PER_WORKER // CHUNK
ROW_UNROLL = 4
NVREG = D_MODEL // 16
MAX_FAM = 512


def _rsqrt(v):
    i = lax.bitcast_convert_type(v, jnp.int32)
    i = jnp.full_like(i, 0x5F3759DF) - lax.shift_right_arithmetic(i, 1)
    y = lax.bitcast_convert_type(i, jnp.float32)
    for _ in range(2):
        y = y * (1.5 - 0.5 * v * y * y)
    return y


_GATHER_DNUMS = lax.GatherDimensionNumbers(
    offset_dims=(), collapsed_slice_dims=(0,), start_index_map=(0,))


def _shuffle(t, perm):
    return lax.gather(t, perm[:, None], dimension_numbers=_GATHER_DNUMS,
                      slice_sizes=(1,),
                      mode=lax.GatherScatterMode.PROMISE_IN_BOUNDS)


def _sc_kernel_body(gid_hbm, fid_hbm, gtab_hbm, ftab_hbm, gamma_hbm, beta_hbm,
                    out_hbm, idx_g, idx_f, rows_ga, rows_gb, ftab_v,
                    out_v0, out_v1, gamma_v, beta_v,
                    sem_ga, sem_gb, sem_ft, sem_o0, sem_o1):
    wid = lax.axis_index("s") * NUM_CORES + lax.axis_index("c")
    base = wid * ROWS_PER_WORKER

    cft = pltpu.async_copy(ftab_hbm, ftab_v, sem_ft)
    cfi = pltpu.async_copy(fid_hbm.at[pl.ds(base, ROWS_PER_WORKER)], idx_f,
                           sem_ft)
    pltpu.sync_copy(gid_hbm.at[pl.ds(base, ROWS_PER_WORKER)], idx_g)
    pltpu.sync_copy(gamma_hbm, gamma_v)
    pltpu.sync_copy(beta_hbm, beta_v)

    gvec = [gamma_v[pl.ds(c * 16, 16)] for c in range(NVREG)]
    bvec = [beta_v[pl.ds(c * 16, 16)] for c in range(NVREG)]

    lane = lax.iota(jnp.int32, 16)
    cols = [lane + c * 16 for c in range(NVREG)]

    def start_gathers(j, rg, sg):
        return pltpu.async_copy(
            gtab_hbm.at[idx_g.at[pl.ds(j * CHUNK, CHUNK)]], rg, sg)

    def compute(j, rows_g, out_v):
        @plsc.parallel_loop(0, CHUNK, step=2, unroll=ROW_UNROLL // 2)
        def body(r0):
            grp = idx_f[pl.ds(pl.multiple_of(j * CHUNK + (r0 & -16), 8), 16)]
            for u in range(2):
                r = r0 + u
                fidv = _shuffle(grp, jnp.full((16,), (r0 & 15) + u,
                                              dtype=jnp.int32))
                frow = lax.shift_right_logical(fidv, 1)
                fcol = (fidv & 1) * D_MODEL
                xs = []
                for c in range(NVREG):
                    g = rows_g[r, pl.ds(c * 16, 16)]
                    f = plsc.load_gather(ftab_v, [frow, fcol + cols[c]])
                    xs.append(g + f)
                t = (xs[0] + xs[1]) + (xs[2] + xs[3])
                q = (xs[0] * xs[0] + xs[1] * xs[1]) + (xs[2] * xs[2] + xs[3] * xs[3])
                s_sum = jnp.sum(t)
                q_sum = jnp.sum(q)
                mean = s_sum * (1.0 / D_MODEL)
                var = q_sum * (1.0 / D_MODEL) - mean * mean
                rv = _rsqrt(var + EPS)
                a = rv
                b = -mean * rv
                for c in range(NVREG):
                    out_v[r, pl.ds(c * 16, 16)] = (
                        (xs[c] * a + b) * gvec[c] + bvec[c])

    def start_out(j, out_v, so):
        return pltpu.async_copy(
            out_v, out_hbm.at[pl.ds(base + j * CHUNK, CHUNK)], so)

    g0 = start_gathers(0, rows_ga, sem_ga)
    g1 = start_gathers(1, rows_gb, sem_gb)
    cft.wait()
    cfi.wait()
    g0.wait()
    compute(0, rows_ga, out_v0)
    o0 = start_out(0, out_v0, sem_o0)
    g2 = start_gathers(2, rows_ga, sem_ga)
    g1.wait()
    compute(1, rows_gb, out_v1)
    o1 = start_out(1, out_v1, sem_o1)
    g3 = start_gathers(3, rows_gb, sem_gb)
    g2.wait()
    o0.wait()
    compute(2, rows_ga, out_v0)
    o2 = start_out(2, out_v0, sem_o0)
    g3.wait()
    o1.wait()
    compute(3, rows_gb, out_v1)
    o3 = start_out(3, out_v1, sem_o1)
    o2.wait()
    o3.wait()


@jax.jit
def _run(gid, fid, gtab_pad, ftab2, gamma, beta):
    mesh = plsc.VectorSubcoreMesh(core_axis_name="c", subcore_axis_name="s")
    k = functools.partial(
        pl.kernel,
        mesh=mesh,
        out_type=jax.ShapeDtypeStruct((BATCH, D_MODEL), jnp.float32),
        compiler_params=pltpu.CompilerParams(use_tc_tiling_on_sc=True,
                                             needs_layout_passes=False),
        scratch_types=[
            pltpu.VMEM((ROWS_PER_WORKER,), jnp.int32),
            pltpu.VMEM((ROWS_PER_WORKER,), jnp.int32),
            pltpu.VMEM((CHUNK, D_PAD), jnp.float32),
            pltpu.VMEM((CHUNK, D_PAD), jnp.float32),
            pltpu.VMEM((MAX_FAM // 2, D_PAD), jnp.float32),
            pltpu.VMEM((CHUNK, D_MODEL), jnp.float32),
            pltpu.VMEM((CHUNK, D_MODEL), jnp.float32),
            pltpu.VMEM((D_MODEL,), jnp.float32),
            pltpu.VMEM((D_MODEL,), jnp.float32),
            pltpu.SemaphoreType.DMA,
            pltpu.SemaphoreType.DMA,
            pltpu.SemaphoreType.DMA,
            pltpu.SemaphoreType.DMA,
            pltpu.SemaphoreType.DMA,
        ],
    )(_sc_kernel_body)
    return k(gid, fid, gtab_pad, ftab2, gamma, beta)


def kernel(game_id, game_family, game_table, family_table, gamma, beta):
    gtab_pad = jnp.pad(game_table, ((0, 0), (0, D_PAD - D_MODEL)))
    ftab2 = family_table.reshape(MAX_FAM // 2, D_PAD)
    return _run(game_id.astype(jnp.int32), game_family.astype(jnp.int32),
                gtab_pad, ftab2, gamma, beta)


# speedup vs baseline: 1.0282x; 1.0282x over previous
import functools

import jax
import jax.numpy as jnp
from jax import lax
from jax.experimental import pallas as pl
from jax.experimental.pallas import tpu as pltpu
from jax.experimental.pallas import tpu_sc as plsc

D_MODEL = 64
D_PAD = 128
BATCH = 16384
EPS = 1e-5

NUM_CORES = 2
NUM_SUBCORES = 16
NUM_WORKERS = NUM_CORES * NUM_SUBCORES
ROWS_PER_WORKER = BATCH // NUM_WORKERS
CHUNK = 128
NUM_CHUNKS = ROWS_PER_WORKER // CHUNK
ROW_UNROLL = 4
NVREG = D_MODEL // 16
MAX_FAM = 512


def _rsqrt(v):
    i = lax.bitcast_convert_type(v, jnp.int32)
    i = jnp.full_like(i, 0x5F3759DF) - lax.shift_right_arithmetic(i, 1)
    y = lax.bitcast_convert_type(i, jnp.float32)
    for _ in range(2):
        y = y * (1.5 - 0.5 * v * y * y)
    return y


_GATHER_DNUMS = lax.GatherDimensionNumbers(
    offset_dims=(), collapsed_slice_dims=(0,), start_index_map=(0,))


def _shuffle(t, perm):
    return lax.gather(t, perm[:, None], dimension_numbers=_GATHER_DNUMS,
                      slice_sizes=(1,),
                      mode=lax.GatherScatterMode.PROMISE_IN_BOUNDS)


def _sc_kernel_body(gid_hbm, fid_hbm, gtab_hbm, ftab_hbm, gamma_hbm, beta_hbm,
                    out_hbm, idx_g, idx_f, rows_ga, rows_gb, ftab_v,
                    out_v0, out_v1, gamma_v, beta_v,
                    sem_ga, sem_gb, sem_ft, sem_o0, sem_o1):
    wid = lax.axis_index("s") * NUM_CORES + lax.axis_index("c")
    base = wid * ROWS_PER_WORKER

    cft = pltpu.async_copy(ftab_hbm, ftab_v, sem_ft)
    cfi = pltpu.async_copy(fid_hbm.at[pl.ds(base, ROWS_PER_WORKER)], idx_f,
                           sem_ft)
    pltpu.sync_copy(gid_hbm.at[pl.ds(base, ROWS_PER_WORKER)], idx_g)
    pltpu.sync_copy(gamma_hbm, gamma_v)
    pltpu.sync_copy(beta_hbm, beta_v)

    gvec = [gamma_v[pl.ds(c * 16, 16)] for c in range(NVREG)]
    bvec = [beta_v[pl.ds(c * 16, 16)] for c in range(NVREG)]

    lane = lax.iota(jnp.int32, 16)
    cols = [lane + c * 16 for c in range(NVREG)]

    def start_gathers(j, rg, sg):
        return pltpu.async_copy(
            gtab_hbm.at[idx_g.at[pl.ds(j * CHUNK, CHUNK)]], rg, sg)

    def compute(j, rows_g, out_v):
        @plsc.parallel_loop(0, CHUNK, step=1, unroll=ROW_UNROLL)
        def body(r):
            grp = idx_f[pl.ds(pl.multiple_of(j * CHUNK + (r & -16), 8), 16)]
            fidv = _shuffle(grp, jnp.full((16,), r & 15, dtype=jnp.int32))
            frow = lax.shift_right_logical(fidv, 1)
            fcol = (fidv & 1) * D_MODEL
            xs = []
            for c in range(NVREG):
                g = rows_g[r, pl.ds(c * 16, 16)]
                f = plsc.load_gather(ftab_v, [frow, fcol + cols[c]])
                xs.append(g + f)
            t = (xs[0] + xs[1]) + (xs[2] + xs[3])
            q = (xs[0] * xs[0] + xs[1] * xs[1]) + (xs[2] * xs[2] + xs[3] * xs[3])
            s_sum = jnp.sum(t)
            q_sum = jnp.sum(q)
            mean = s_sum * (1.0 / D_MODEL)
            var = q_sum * (1.0 / D_MODEL) - mean * mean
            rv = _rsqrt(var + EPS)
            a = rv
            b = -mean * rv
            for c in range(NVREG):
                out_v[r, pl.ds(c * 16, 16)] = (xs[c] * a + b) * gvec[c] + bvec[c]

    def start_out(j, out_v, so):
        return pltpu.async_copy(
            out_v, out_hbm.at[pl.ds(base + j * CHUNK, CHUNK)], so)

    g0 = start_gathers(0, rows_ga, sem_ga)
    g1 = start_gathers(1, rows_gb, sem_gb)
    cft.wait()
    cfi.wait()
    g0.wait()
    compute(0, rows_ga, out_v0)
    o0 = start_out(0, out_v0, sem_o0)
    g2 = start_gathers(2, rows_ga, sem_ga)
    g1.wait()
    compute(1, rows_gb, out_v1)
    o1 = start_out(1, out_v1, sem_o1)
    g3 = start_gathers(3, rows_gb, sem_gb)
    g2.wait()
    o0.wait()
    compute(2, rows_ga, out_v0)
    o2 = start_out(2, out_v0, sem_o0)
    g3.wait()
    o1.wait()
    compute(3, rows_gb, out_v1)
    o3 = start_out(3, out_v1, sem_o1)
    o2.wait()
    o3.wait()


@jax.jit
def _run(gid, fid, gtab_pad, ftab2, gamma, beta):
    mesh = plsc.VectorSubcoreMesh(core_axis_name="c", subcore_axis_name="s")
    k = functools.partial(
        pl.kernel,
        mesh=mesh,
        out_type=jax.ShapeDtypeStruct((BATCH, D_MODEL), jnp.float32),
        compiler_params=pltpu.CompilerParams(use_tc_tiling_on_sc=True,
                                             needs_layout_passes=False),
        scratch_types=[
            pltpu.VMEM((ROWS_PER_WORKER,), jnp.int32),
            pltpu.VMEM((ROWS_PER_WORKER,), jnp.int32),
            pltpu.VMEM((CHUNK, D_PAD), jnp.float32),
            pltpu.VMEM((CHUNK, D_PAD), jnp.float32),
            pltpu.VMEM((MAX_FAM // 2, D_PAD), jnp.float32),
            pltpu.VMEM((CHUNK, D_MODEL), jnp.float32),
            pltpu.VMEM((CHUNK, D_MODEL), jnp.float32),
            pltpu.VMEM((D_MODEL,), jnp.float32),
            pltpu.VMEM((D_MODEL,), jnp.float32),
            pltpu.SemaphoreType.DMA,
            pltpu.SemaphoreType.DMA,
            pltpu.SemaphoreType.DMA,
            pltpu.SemaphoreType.DMA,
            pltpu.SemaphoreType.DMA,
        ],
    )(_sc_kernel_body)
    return k(gid, fid, gtab_pad, ftab2, gamma, beta)


def kernel(game_id, game_family, game_table, family_table, gamma, beta):
    gtab_pad = jnp.pad(game_table, ((0, 0), (0, D_PAD - D_MODEL)))
    ftab2 = family_table.reshape(MAX_FAM // 2, D_PAD)
    return _run(game_id.astype(jnp.int32), game_family.astype(jnp.int32),
                gtab_pad, ftab2, gamma, beta)


# speedup vs baseline: 1.0391x; 1.0106x over previous
import functools

import jax
import jax.numpy as jnp
from jax import lax
from jax.experimental import pallas as pl
from jax.experimental.pallas import tpu as pltpu
from jax.experimental.pallas import tpu_sc as plsc

D_MODEL = 64
D_PAD = 128
BATCH = 16384
EPS = 1e-5

NUM_CORES = 2
NUM_SUBCORES = 16
NUM_WORKERS = NUM_CORES * NUM_SUBCORES
ROWS_PER_WORKER = BATCH // NUM_WORKERS
CHUNK = 128
NUM_CHUNKS = ROWS_PER_WORKER // CHUNK
ROW_UNROLL = 4
NVREG = D_MODEL // 16
MAX_FAM = 512


def _rsqrt(v):
    i = lax.bitcast_convert_type(v, jnp.int32)
    i = jnp.full_like(i, 0x5F3759DF) - lax.shift_right_arithmetic(i, 1)
    y = lax.bitcast_convert_type(i, jnp.float32)
    for _ in range(1):
        y = y * (1.5 - 0.5 * v * y * y)
    return y


_GATHER_DNUMS = lax.GatherDimensionNumbers(
    offset_dims=(), collapsed_slice_dims=(0,), start_index_map=(0,))


def _shuffle(t, perm):
    return lax.gather(t, perm[:, None], dimension_numbers=_GATHER_DNUMS,
                      slice_sizes=(1,),
                      mode=lax.GatherScatterMode.PROMISE_IN_BOUNDS)


def _sc_kernel_body(gid_hbm, fid_hbm, gtab_hbm, ftab_hbm, gamma_hbm, beta_hbm,
                    out_hbm, idx_g, idx_f, rows_ga, rows_gb, ftab_v,
                    out_v0, out_v1, gamma_v, beta_v,
                    sem_ga, sem_gb, sem_ft, sem_o0, sem_o1):
    wid = lax.axis_index("s") * NUM_CORES + lax.axis_index("c")
    base = wid * ROWS_PER_WORKER

    cft = pltpu.async_copy(ftab_hbm, ftab_v, sem_ft)
    cfi = pltpu.async_copy(fid_hbm.at[pl.ds(base, ROWS_PER_WORKER)], idx_f,
                           sem_ft)
    pltpu.sync_copy(gid_hbm.at[pl.ds(base, ROWS_PER_WORKER)], idx_g)
    pltpu.sync_copy(gamma_hbm, gamma_v)
    pltpu.sync_copy(beta_hbm, beta_v)

    gvec = [gamma_v[pl.ds(c * 16, 16)] for c in range(NVREG)]
    bvec = [beta_v[pl.ds(c * 16, 16)] for c in range(NVREG)]

    lane = lax.iota(jnp.int32, 16)
    cols = [lane + c * 16 for c in range(NVREG)]

    def start_gathers(j, rg, sg):
        return pltpu.async_copy(
            gtab_hbm.at[idx_g.at[pl.ds(j * CHUNK, CHUNK)]], rg, sg)

    def compute(j, rows_g, out_v):
        @plsc.parallel_loop(0, CHUNK, step=1, unroll=ROW_UNROLL)
        def body(r):
            grp = idx_f[pl.ds(pl.multiple_of(j * CHUNK + (r & -16), 8), 16)]
            fidv = _shuffle(grp, jnp.full((16,), r & 15, dtype=jnp.int32))
            frow = lax.shift_right_logical(fidv, 1)
            fcol = (fidv & 1) * D_MODEL
            xs = []
            for c in range(NVREG):
                g = rows_g[r, pl.ds(c * 16, 16)]
                f = plsc.load_gather(ftab_v, [frow, fcol + cols[c]])
                xs.append(g + f)
            t = (xs[0] + xs[1]) + (xs[2] + xs[3])
            q = (xs[0] * xs[0] + xs[1] * xs[1]) + (xs[2] * xs[2] + xs[3] * xs[3])
            s_sum = jnp.sum(t)
            q_sum = jnp.sum(q)
            mean = s_sum * (1.0 / D_MODEL)
            var = q_sum * (1.0 / D_MODEL) - mean * mean
            rv = _rsqrt(var + EPS)
            a = rv
            b = -mean * rv
            for c in range(NVREG):
                out_v[r, pl.ds(c * 16, 16)] = (xs[c] * a + b) * gvec[c] + bvec[c]

    def start_out(j, out_v, so):
        return pltpu.async_copy(
            out_v, out_hbm.at[pl.ds(base + j * CHUNK, CHUNK)], so)

    g0 = start_gathers(0, rows_ga, sem_ga)
    g1 = start_gathers(1, rows_gb, sem_gb)
    cft.wait()
    cfi.wait()
    g0.wait()
    compute(0, rows_ga, out_v0)
    o0 = start_out(0, out_v0, sem_o0)
    g2 = start_gathers(2, rows_ga, sem_ga)
    g1.wait()
    compute(1, rows_gb, out_v1)
    o1 = start_out(1, out_v1, sem_o1)
    g3 = start_gathers(3, rows_gb, sem_gb)
    g2.wait()
    o0.wait()
    compute(2, rows_ga, out_v0)
    o2 = start_out(2, out_v0, sem_o0)
    g3.wait()
    o1.wait()
    compute(3, rows_gb, out_v1)
    o3 = start_out(3, out_v1, sem_o1)
    o2.wait()
    o3.wait()


@jax.jit
def _run(gid, fid, gtab_pad, ftab2, gamma, beta):
    mesh = plsc.VectorSubcoreMesh(core_axis_name="c", subcore_axis_name="s")
    k = functools.partial(
        pl.kernel,
        mesh=mesh,
        out_type=jax.ShapeDtypeStruct((BATCH, D_MODEL), jnp.float32),
        compiler_params=pltpu.CompilerParams(use_tc_tiling_on_sc=True,
                                             needs_layout_passes=False),
        scratch_types=[
            pltpu.VMEM((ROWS_PER_WORKER,), jnp.int32),
            pltpu.VMEM((ROWS_PER_WORKER,), jnp.int32),
            pltpu.VMEM((CHUNK, D_PAD), jnp.float32),
            pltpu.VMEM((CHUNK, D_PAD), jnp.float32),
            pltpu.VMEM((MAX_FAM // 2, D_PAD), jnp.float32),
            pltpu.VMEM((CHUNK, D_MODEL), jnp.float32),
            pltpu.VMEM((CHUNK, D_MODEL), jnp.float32),
            pltpu.VMEM((D_MODEL,), jnp.float32),
            pltpu.VMEM((D_MODEL,), jnp.float32),
            pltpu.SemaphoreType.DMA,
            pltpu.SemaphoreType.DMA,
            pltpu.SemaphoreType.DMA,
            pltpu.SemaphoreType.DMA,
            pltpu.SemaphoreType.DMA,
        ],
    )(_sc_kernel_body)
    return k(gid, fid, gtab_pad, ftab2, gamma, beta)


def kernel(game_id, game_family, game_table, family_table, gamma, beta):
    gtab_pad = jnp.pad(game_table, ((0, 0), (0, D_PAD - D_MODEL)))
    ftab2 = family_table.reshape(MAX_FAM // 2, D_PAD)
    return _run(game_id.astype(jnp.int32), game_family.astype(jnp.int32),
                gtab_pad, ftab2, gamma, beta)
